# speedup vs baseline: 1.1719x; 1.1719x over previous
import functools

import jax
import jax.numpy as jnp
from jax import lax
from jax.experimental import pallas as pl
from jax.experimental.pallas import tpu as pltpu
from jax.experimental.pallas import tpu_sc as plsc

_RATIO = 1.703
_LANES = 16


@functools.lru_cache(maxsize=None)
def _build(batch: int):
    info = plsc.get_sparse_core_info()
    num_workers = info.num_cores * info.num_subcores
    assert batch % (num_workers * _LANES) == 0
    b_per_w = batch // num_workers
    mesh = plsc.VectorSubcoreMesh(core_axis_name="c", subcore_axis_name="s")

    out_t = jax.ShapeDtypeStruct((batch,), jnp.float32)

    @functools.partial(
        pl.kernel,
        mesh=mesh,
        out_type=[out_t, out_t, out_t],
        scratch_types=[
            pltpu.VMEM((b_per_w,), jnp.int32),
            pltpu.VMEM((b_per_w,), jnp.int32),
            pltpu.VMEM((b_per_w,), jnp.float32),
            pltpu.VMEM((b_per_w,), jnp.float32),
            pltpu.VMEM((b_per_w,), jnp.float32),
            pltpu.VMEM((b_per_w,), jnp.float32),
            pltpu.SemaphoreType.DMA,
            pltpu.SemaphoreType.DMA,
            pltpu.SemaphoreType.DMA,
        ],
    )
    def irt(stu_id_hbm, exer_id_hbm, theta_tab, disc_tab, diff_tab,
            out_hbm, b_hbm, theta_hbm,
            sidx_v, eidx_v, theta_v, disc_v, diff_v, out_v,
            sem0, sem1, sem2):
        wid = lax.axis_index("s") * info.num_cores + lax.axis_index("c")
        base = wid * b_per_w
        pltpu.sync_copy(stu_id_hbm.at[pl.ds(base, b_per_w)], sidx_v)
        pltpu.sync_copy(exer_id_hbm.at[pl.ds(base, b_per_w)], eidx_v)
        g0 = pltpu.async_copy(theta_tab.at[sidx_v], theta_v, sem0)
        g1 = pltpu.async_copy(disc_tab.at[eidx_v], disc_v, sem1)
        g2 = pltpu.async_copy(diff_tab.at[eidx_v], diff_v, sem2)
        g0.wait()
        g1.wait()
        g2.wait()

        def body(i, _):
            off = pl.multiple_of(i * _LANES, _LANES)
            th = theta_v[pl.ds(off, _LANES)]
            dc = disc_v[pl.ds(off, _LANES)]
            df = diff_v[pl.ds(off, _LANES)]
            a = 1.0 / (1.0 + jnp.exp(-dc))
            out_v[pl.ds(off, _LANES)] = 1.0 / (
                1.0 + jnp.exp(-_RATIO * a * (th - df)))
            return 0

        lax.fori_loop(0, b_per_w // _LANES, body, 0)
        pltpu.sync_copy(out_v, out_hbm.at[pl.ds(base, b_per_w)])
        pltpu.sync_copy(diff_v, b_hbm.at[pl.ds(base, b_per_w)])
        pltpu.sync_copy(theta_v, theta_hbm.at[pl.ds(base, b_per_w)])

    return irt


def kernel(stu_id, exer_id, student_emb, proj_disc, proj_diff):
    batch = stu_id.shape[0]
    irt = _build(batch)
    out, b, theta = irt(
        stu_id,
        exer_id,
        student_emb.reshape(-1),
        proj_disc.reshape(-1),
        proj_diff.reshape(-1),
    )
    return out, b.reshape(batch, 1), theta.reshape(batch, 1)


# speedup vs baseline: 1.1814x; 1.0081x over previous
import functools

import jax
import jax.numpy as jnp
from jax import lax
from jax.experimental import pallas as pl
from jax.experimental.pallas import tpu as pltpu
from jax.experimental.pallas import tpu_sc as plsc

_RATIO = 1.703
_LANES = 16


@functools.lru_cache(maxsize=None)
def _build(batch: int):
    info = plsc.get_sparse_core_info()
    num_workers = info.num_cores * info.num_subcores
    assert batch % (num_workers * _LANES) == 0
    b_per_w = batch // num_workers
    mesh = plsc.VectorSubcoreMesh(core_axis_name="c", subcore_axis_name="s")

    out_t = jax.ShapeDtypeStruct((batch,), jnp.float32)

    @functools.partial(
        pl.kernel,
        mesh=mesh,
        out_type=[out_t, out_t, out_t],
        scratch_types=[
            pltpu.VMEM((b_per_w,), jnp.int32),
            pltpu.VMEM((b_per_w,), jnp.int32),
            pltpu.VMEM((b_per_w,), jnp.float32),
            pltpu.VMEM((b_per_w,), jnp.float32),
            pltpu.VMEM((b_per_w,), jnp.float32),
            pltpu.VMEM((b_per_w,), jnp.float32),
            pltpu.SemaphoreType.DMA,
            pltpu.SemaphoreType.DMA,
            pltpu.SemaphoreType.DMA,
        ],
    )
    def irt(stu_id_hbm, exer_id_hbm, theta_tab, disc_tab, diff_tab,
            out_hbm, b_hbm, theta_hbm,
            sidx_v, eidx_v, theta_f, disc_f, diff_f, out_v,
            sem0, sem1, sem2):
        wid = lax.axis_index("s") * info.num_cores + lax.axis_index("c")
        base = wid * b_per_w
        sl = pl.ds(base, b_per_w)
        c0 = pltpu.async_copy(stu_id_hbm.at[sl], sidx_v, sem0)
        c1 = pltpu.async_copy(exer_id_hbm.at[sl], eidx_v, sem1)
        c1.wait()
        g1 = pltpu.async_copy(disc_tab.at[eidx_v], disc_f, sem1)
        g2 = pltpu.async_copy(diff_tab.at[eidx_v], diff_f, sem2)
        c0.wait()
        g0 = pltpu.async_copy(theta_tab.at[sidx_v], theta_f, sem0)
        g1.wait()
        g2.wait()
        g0.wait()

        def body(i, _):
            csl = pl.ds(pl.multiple_of(i * _LANES, _LANES), _LANES)
            th = theta_f[csl]
            dc = disc_f[csl]
            df = diff_f[csl]
            a = 1.0 / (1.0 + jnp.exp(-dc))
            out_v[csl] = 1.0 / (1.0 + jnp.exp(-_RATIO * a * (th - df)))
            return 0

        lax.fori_loop(0, b_per_w // _LANES, body, 0)
        w0 = pltpu.async_copy(out_v, out_hbm.at[sl], sem0)
        w1 = pltpu.async_copy(diff_f, b_hbm.at[sl], sem1)
        w2 = pltpu.async_copy(theta_f, theta_hbm.at[sl], sem2)
        w0.wait()
        w1.wait()
        w2.wait()

    return irt


def kernel(stu_id, exer_id, student_emb, proj_disc, proj_diff):
    batch = stu_id.shape[0]
    irt = _build(batch)
    out, b, theta = irt(
        stu_id,
        exer_id,
        student_emb.reshape(-1),
        proj_disc.reshape(-1),
        proj_diff.reshape(-1),
    )
    return out, b.reshape(batch, 1), theta.reshape(batch, 1)


# speedup vs baseline: 1.1825x; 1.0009x over previous
import functools

import jax
import jax.numpy as jnp
from jax import lax
from jax.experimental import pallas as pl
from jax.experimental.pallas import tpu as pltpu
from jax.experimental.pallas import tpu_sc as plsc

_RATIO = 1.703
_LANES = 16


@functools.lru_cache(maxsize=None)
def _build(batch: int):
    info = plsc.get_sparse_core_info()
    num_workers = info.num_cores * info.num_subcores
    assert batch % (num_workers * _LANES) == 0
    b_per_w = batch // num_workers
    mesh = plsc.VectorSubcoreMesh(core_axis_name="c", subcore_axis_name="s")

    out_t = jax.ShapeDtypeStruct((batch,), jnp.float32)

    @functools.partial(
        pl.kernel,
        mesh=mesh,
        out_type=[out_t, out_t, out_t],
        scratch_types=[
            pltpu.VMEM((b_per_w,), jnp.int32),
            pltpu.VMEM((b_per_w,), jnp.int32),
            pltpu.VMEM((b_per_w,), jnp.float32),
            pltpu.VMEM((b_per_w,), jnp.float32),
            pltpu.VMEM((b_per_w,), jnp.float32),
            pltpu.VMEM((b_per_w,), jnp.float32),
            pltpu.SemaphoreType.DMA,
            pltpu.SemaphoreType.DMA,
            pltpu.SemaphoreType.DMA,
        ],
    )
    def irt(stu_id_hbm, exer_id_hbm, theta_tab, disc_tab, diff_tab,
            out_hbm, b_hbm, theta_hbm,
            sidx_v, eidx_v, theta_f, disc_f, diff_f, out_v,
            sem0, sem1, sem2):
        wid = lax.axis_index("s") * info.num_cores + lax.axis_index("c")
        base = wid * b_per_w
        sl = pl.ds(base, b_per_w)
        c0 = pltpu.async_copy(stu_id_hbm.at[sl], sidx_v, sem0)
        c1 = pltpu.async_copy(exer_id_hbm.at[sl], eidx_v, sem1)
        c0.wait()
        g0 = pltpu.async_copy(theta_tab.at[sidx_v], theta_f, sem0)
        c1.wait()
        g1 = pltpu.async_copy(disc_tab.at[eidx_v], disc_f, sem1)
        g2 = pltpu.async_copy(diff_tab.at[eidx_v], diff_f, sem2)
        g0.wait()
        w2 = pltpu.async_copy(theta_f, theta_hbm.at[sl], sem0)
        g2.wait()
        w1 = pltpu.async_copy(diff_f, b_hbm.at[sl], sem2)
        g1.wait()

        def body(i, _):
            csl = pl.ds(pl.multiple_of(i * _LANES, _LANES), _LANES)
            th = theta_f[csl]
            dc = disc_f[csl]
            df = diff_f[csl]
            a = 1.0 / (1.0 + jnp.exp(-dc))
            out_v[csl] = 1.0 / (1.0 + jnp.exp(-_RATIO * a * (th - df)))
            return 0

        lax.fori_loop(0, b_per_w // _LANES, body, 0)
        w0 = pltpu.async_copy(out_v, out_hbm.at[sl], sem1)
        w0.wait()
        w1.wait()
        w2.wait()

    return irt


def kernel(stu_id, exer_id, student_emb, proj_disc, proj_diff):
    batch = stu_id.shape[0]
    irt = _build(batch)
    out, b, theta = irt(
        stu_id,
        exer_id,
        student_emb.reshape(-1),
        proj_disc.reshape(-1),
        proj_diff.reshape(-1),
    )
    return out, b.reshape(batch, 1), theta.reshape(batch, 1)


# speedup vs baseline: 1.2126x; 1.0255x over previous
import functools

import jax
import jax.numpy as jnp
from jax import lax
from jax.experimental import pallas as pl
from jax.experimental.pallas import tpu as pltpu
from jax.experimental.pallas import tpu_sc as plsc

_RATIO = 1.703
_LANES = 16


@functools.lru_cache(maxsize=None)
def _build(batch: int):
    info = plsc.get_sparse_core_info()
    num_workers = info.num_cores * info.num_subcores
    assert batch % (num_workers * _LANES) == 0
    b_per_w = batch // num_workers
    mesh = plsc.VectorSubcoreMesh(core_axis_name="c", subcore_axis_name="s")

    out_t = jax.ShapeDtypeStruct((batch,), jnp.float32)

    @functools.partial(
        pl.kernel,
        mesh=mesh,
        out_type=[out_t, out_t],
        scratch_types=[
            pltpu.VMEM((b_per_w,), jnp.int32),
            pltpu.VMEM((b_per_w,), jnp.float32),
            pltpu.VMEM((b_per_w,), jnp.float32),
            pltpu.VMEM((b_per_w,), jnp.float32),
            pltpu.SemaphoreType.DMA,
            pltpu.SemaphoreType.DMA,
        ],
    )
    def irt_a(exer_id_hbm, disc_tab, diff_tab,
              b_hbm, s_hbm,
              eidx_v, disc_f, diff_f, s_v,
              sem0, sem1):
        wid = lax.axis_index("s") * info.num_cores + lax.axis_index("c")
        sl = pl.ds(wid * b_per_w, b_per_w)
        pltpu.sync_copy(exer_id_hbm.at[sl], eidx_v)
        g1 = pltpu.async_copy(disc_tab.at[eidx_v], disc_f, sem0)
        g2 = pltpu.async_copy(diff_tab.at[eidx_v], diff_f, sem1)
        g2.wait()
        w1 = pltpu.async_copy(diff_f, b_hbm.at[sl], sem1)
        g1.wait()

        def body(i, _):
            csl = pl.ds(pl.multiple_of(i * _LANES, _LANES), _LANES)
            dc = disc_f[csl]
            s_v[csl] = _RATIO / (1.0 + jnp.exp(-dc))
            return 0

        lax.fori_loop(0, b_per_w // _LANES, body, 0)
        w2 = pltpu.async_copy(s_v, s_hbm.at[sl], sem0)
        w1.wait()
        w2.wait()

    @functools.partial(
        pl.kernel,
        mesh=mesh,
        out_type=[out_t, out_t],
        scratch_types=[
            pltpu.VMEM((b_per_w,), jnp.int32),
            pltpu.VMEM((b_per_w,), jnp.float32),
            pltpu.VMEM((b_per_w,), jnp.float32),
            pltpu.VMEM((b_per_w,), jnp.float32),
            pltpu.VMEM((b_per_w,), jnp.float32),
            pltpu.SemaphoreType.DMA,
            pltpu.SemaphoreType.DMA,
            pltpu.SemaphoreType.DMA,
        ],
    )
    def irt_b(stu_id_hbm, theta_tab, s_hbm, b_hbm,
              out_hbm, theta_hbm,
              sidx_v, theta_f, s_f, b_f, out_v,
              sem0, sem1, sem2):
        wid = lax.axis_index("s") * info.num_cores + lax.axis_index("c")
        sl = pl.ds(wid * b_per_w, b_per_w)
        c0 = pltpu.async_copy(stu_id_hbm.at[sl], sidx_v, sem0)
        c1 = pltpu.async_copy(s_hbm.at[sl], s_f, sem1)
        c2 = pltpu.async_copy(b_hbm.at[sl], b_f, sem2)
        c0.wait()
        g0 = pltpu.async_copy(theta_tab.at[sidx_v], theta_f, sem0)
        g0.wait()
        w1 = pltpu.async_copy(theta_f, theta_hbm.at[sl], sem0)
        c1.wait()
        c2.wait()

        def body(i, _):
            csl = pl.ds(pl.multiple_of(i * _LANES, _LANES), _LANES)
            th = theta_f[csl]
            s = s_f[csl]
            bb = b_f[csl]
            out_v[csl] = 1.0 / (1.0 + jnp.exp(-s * (th - bb)))
            return 0

        lax.fori_loop(0, b_per_w // _LANES, body, 0)
        w0 = pltpu.async_copy(out_v, out_hbm.at[sl], sem1)
        w0.wait()
        w1.wait()

    return irt_a, irt_b


def kernel(stu_id, exer_id, student_emb, proj_disc, proj_diff):
    batch = stu_id.shape[0]
    irt_a, irt_b = _build(batch)
    b, s = irt_a(exer_id, proj_disc.reshape(-1), proj_diff.reshape(-1))
    out, theta = irt_b(stu_id, student_emb.reshape(-1), s, b)
    return out, b.reshape(batch, 1), theta.reshape(batch, 1)
